# baseline (speedup 1.0000x reference)
import functools

import jax
import jax.numpy as jnp
from jax import lax
from jax.experimental import pallas as pl
from jax.experimental.pallas import tpu as pltpu
from jax.experimental.pallas import tpu_sc as plsc

F32 = jnp.float32
EPS = 1e-7
LN_EPS = 1e-5
DEN_EPS = 1e-16

NSUB = 16
LANES = 16
CH = 80


def _sc_aggr_kernel(N_PAD, E, H, gather_input):
    H2 = 2 * H
    H4 = 4 * H
    EW = E // NSUB
    NCH = EW // CH
    NCHP = ((NCH + 7) // 8) * 8
    ROWS_W = N_PAD // NSUB

    mesh = plsc.VectorSubcoreMesh(core_axis_name="c", subcore_axis_name="s",
                                  num_cores=1)

    out_type = [
        jax.ShapeDtypeStruct((E, H2), F32),
        jax.ShapeDtypeStruct((E, H2), F32),
        jax.ShapeDtypeStruct((E, H2), F32),
        jax.ShapeDtypeStruct((N_PAD, H4), F32),
    ]
    scratch = [
        pltpu.VMEM((NCHP, CH), jnp.int32),
        pltpu.VMEM((NCHP, CH), jnp.int32),
        pltpu.VMEM((2, CH, H), F32),
        pltpu.VMEM((2, CH, H), F32),
        pltpu.VMEM((2, CH, H4), F32),
        pltpu.VMEM((2, CH, H2), F32),
        pltpu.VMEM((2, CH, H4), F32),
        pltpu.VMEM((CH, H4), F32),
        pltpu.VMEM((LANES,), F32),
        pltpu.VMEM_SHARED((N_PAD, H4), F32),
        pltpu.SemaphoreType.DMA((2,)),
        pltpu.SemaphoreType.DMA((2,)),
        pltpu.SemaphoreType.DMA((2,)),
        pltpu.SemaphoreType.DMA((2,)),
        pltpu.SemaphoreType.DMA((2,)),
    ]

    @functools.partial(pl.kernel, out_type=out_type, mesh=mesh,
                       scratch_types=scratch,
                       compiler_params=pltpu.CompilerParams(
                           use_tc_tiling_on_sc=False))
    def body(h_hbm, row2d, col2d, tvec, xl0_hbm, den_hbm, num_hbm, dn_hbm,
             rowv, colv, hr_v, hc_v, contrib, xlbuf,
             gbuf, zbuf, tv_v, acc, semg, semc, sems, semo, semo2):
        s = lax.axis_index("s")
        ebase = s * EW

        z16 = jnp.zeros((LANES,), F32)

        def zrow(i, carry):
            for k in range(4):
                zbuf[i, pl.ds(k * LANES, LANES)] = z16
            return carry

        lax.fori_loop(0, CH, zrow, 0, unroll=8)
        for z in range(ROWS_W // CH):
            pltpu.sync_copy(zbuf, acc.at[pl.ds(s * ROWS_W + z * CH, CH)])

        pltpu.sync_copy(row2d.at[pl.ds(s * NCHP, NCHP)], rowv)
        pltpu.sync_copy(col2d.at[pl.ds(s * NCHP, NCHP)], colv)
        pltpu.sync_copy(tvec, tv_v)
        tv = tv_v[...]
        epsv = jnp.full((LANES,), EPS, F32)

        plsc.subcore_barrier()

        def issue_in(j, p):
            if gather_input:
                pltpu.async_copy(h_hbm.at[rowv.at[j]], hr_v.at[p], semg.at[p])
                pltpu.async_copy(h_hbm.at[colv.at[j]], hc_v.at[p], semc.at[p])
            else:
                pltpu.async_copy(h_hbm.at[pl.ds(ebase + j * CH, CH)],
                                 xlbuf.at[p], semg.at[p])

        def wait_in(j, p):
            if gather_input:
                pltpu.make_async_copy(h_hbm.at[rowv.at[j]], hr_v.at[p],
                                      semg.at[p]).wait()
                pltpu.make_async_copy(h_hbm.at[colv.at[j]], hc_v.at[p],
                                      semc.at[p]).wait()
            else:
                pltpu.make_async_copy(h_hbm.at[pl.ds(ebase + j * CH, CH)],
                                      xlbuf.at[p], semg.at[p]).wait()

        def wait_scat(j, p):
            pltpu.make_async_copy(contrib.at[p], acc.at[colv.at[j]],
                                  sems.at[p]).wait()
            if gather_input:
                pltpu.make_async_copy(
                    xlbuf.at[p], xl0_hbm.at[pl.ds(ebase + j * CH, CH)],
                    semo.at[p]).wait()

        issue_in(0, 0)

        def chunk(j, carry):
            p = lax.rem(j, 2)
            q = 1 - p

            @pl.when(j + 1 < NCH)
            def _():
                issue_in(j + 1, q)

            wait_in(j, p)

            @pl.when(j >= 2)
            def _():
                wait_scat(j, p)

            def edge(e, ecarry):
                if gather_input:
                    hr = hr_v[p, e]
                    hc = hc_v[p, e]
                    mr = jnp.maximum(hr, 0.0) + epsv
                    mc = jnp.maximum(hc, 0.0) + epsv
                    xlbuf[p, e, pl.ds(0, LANES)] = hr
                    xlbuf[p, e, pl.ds(LANES, LANES)] = hc
                else:
                    mr = xlbuf[p, e, pl.ds(0, LANES)] + epsv
                    mc = xlbuf[p, e, pl.ds(LANES, LANES)] + epsv
                er = jnp.exp(mr * tv)
                ec = jnp.exp(mc * tv)
                contrib[p, e, pl.ds(0, LANES)] = er
                contrib[p, e, pl.ds(LANES, LANES)] = ec
                contrib[p, e, pl.ds(2 * LANES, LANES)] = mr * er
                contrib[p, e, pl.ds(3 * LANES, LANES)] = mc * ec
                return ecarry

            lax.fori_loop(0, CH, edge, 0, unroll=4)
            pltpu.async_copy(contrib.at[p], acc.at[colv.at[j]], sems.at[p],
                             add=True)
            if gather_input:
                pltpu.async_copy(
                    xlbuf.at[p], xl0_hbm.at[pl.ds(ebase + j * CH, CH)],
                    semo.at[p])
            return carry

        lax.fori_loop(0, NCH, chunk, 0)
        wait_scat(NCH - 2, (NCH - 2) % 2)
        wait_scat(NCH - 1, (NCH - 1) % 2)
        plsc.subcore_barrier()

        nslice = pl.ds(s * ROWS_W, ROWS_W)
        pltpu.sync_copy(acc.at[nslice], dn_hbm.at[nslice])
        plsc.subcore_barrier()

        def issue_g(j, p):
            pltpu.async_copy(dn_hbm.at[rowv.at[j]], gbuf.at[p], semg.at[p])

        def wait_g(j, p):
            pltpu.make_async_copy(dn_hbm.at[rowv.at[j]], gbuf.at[p],
                                  semg.at[p]).wait()

        def wait_out(j, p):
            e_sl = pl.ds(ebase + j * CH, CH)
            pltpu.make_async_copy(
                gbuf.at[p, :, pl.ds(0, H2)], den_hbm.at[e_sl],
                semo.at[p]).wait()
            pltpu.make_async_copy(
                gbuf.at[p, :, pl.ds(H2, H2)], num_hbm.at[e_sl],
                semo2.at[p]).wait()

        issue_g(0, 0)

        def g(j, carry):
            p = lax.rem(j, 2)
            q = 1 - p

            @pl.when(j >= 1)
            def _():
                wait_out(j - 1, q)

            @pl.when(j + 1 < NCH)
            def _():
                issue_g(j + 1, q)

            wait_g(j, p)
            e_sl = pl.ds(ebase + j * CH, CH)
            pltpu.async_copy(gbuf.at[p, :, pl.ds(0, H2)], den_hbm.at[e_sl],
                             semo.at[p])
            pltpu.async_copy(gbuf.at[p, :, pl.ds(H2, H2)], num_hbm.at[e_sl],
                             semo2.at[p])
            return carry

        lax.fori_loop(0, NCH, g, 0)
        wait_out(NCH - 1, (NCH - 1) % 2)

    return body


def _layer_norm_block(z, g, b):
    k = z.shape[-1]
    w = jnp.full((k, k), 1.0 / k, F32)
    mu = jnp.dot(z, w, preferred_element_type=F32)
    d = z - mu
    var = jnp.dot(d * d, w, preferred_element_type=F32)
    return d * lax.rsqrt(var + LN_EPS) * g + b


def _enc_body(x_ref, w_ref, b_ref, o_ref):
    o_ref[...] = (
        jnp.dot(x_ref[...], w_ref[...], preferred_element_type=F32)
        + b_ref[...]
    )


def _mid_body(xl0_ref, den_ref, num_ref, w1_ref, b1_ref, g1_ref, be1_ref,
              w2_ref, b2_ref, ng_ref, nb_ref, xl1_ref, hh_ref):
    aggr = num_ref[...] / (den_ref[...] + DEN_EPS)
    out = aggr + xl0_ref[...]
    z = jnp.dot(out, w1_ref[...], preferred_element_type=F32) + b1_ref[...]
    z = _layer_norm_block(z, g1_ref[...], be1_ref[...])
    z = jnp.maximum(z, 0.0)
    xl1 = jnp.dot(z, w2_ref[...], preferred_element_type=F32) + b2_ref[...]
    xl1_ref[...] = xl1
    hh_ref[...] = jnp.maximum(_layer_norm_block(xl1, ng_ref[...], nb_ref[...]),
                              0.0)


def _final_body(hh_ref, xl1_ref, den_ref, num_ref, w1_ref, b1_ref, g1_ref,
                be1_ref, w2_ref, b2_ref, n0g_ref, n0b_ref, wl_ref, bl_ref,
                y_ref):
    aggr = num_ref[...] / (den_ref[...] + DEN_EPS)
    out = aggr + hh_ref[...]
    z = jnp.dot(out, w1_ref[...], preferred_element_type=F32) + b1_ref[...]
    z = _layer_norm_block(z, g1_ref[...], be1_ref[...])
    z = jnp.maximum(z, 0.0)
    z = jnp.dot(z, w2_ref[...], preferred_element_type=F32) + b2_ref[...]
    xl2 = xl1_ref[...] + z
    q = jnp.maximum(_layer_norm_block(xl2, n0g_ref[...], n0b_ref[...]), 0.0)
    y_ref[...] = (
        jnp.dot(q, wl_ref[...], preferred_element_type=F32) + bl_ref[...]
    )


def kernel(x, edge_index, edge_attr, W_enc, b_enc, t0, W1_0, b1_0, g1_0,
           be1_0, W2_0, b2_0, t1, W1_1, b1_1, g1_1, be1_1, W2_1, b2_1,
           n0_g, n0_b, n1_g, n1_b, W_lin, b_lin):
    N, F_in = x.shape
    E = edge_index.shape[1]
    H = W_enc.shape[1]
    H2 = 2 * H
    H4 = 4 * H
    Hm = W1_0.shape[1]
    F_out = W_lin.shape[1]

    N_PAD = ((N + NSUB * CH - 1) // (NSUB * CH)) * (NSUB * CH)

    nch = (E // NSUB) // CH
    nchp = ((nch + 7) // 8) * 8

    def pad_idx(v):
        v3 = v.astype(jnp.int32).reshape(NSUB, nch, CH)
        v3 = jnp.pad(v3, ((0, 0), (0, nchp - nch), (0, 0)))
        return v3.reshape(NSUB * nchp, CH)

    row2d = pad_idx(edge_index[0])
    col2d = pad_idx(edge_index[1])
    t0v = jnp.full((LANES,), t0, F32)
    t1v = jnp.full((LANES,), t1, F32)

    h = pl.pallas_call(
        _enc_body,
        out_shape=jax.ShapeDtypeStruct((N_PAD, H), F32),
        in_specs=[
            pl.BlockSpec((N_PAD, F_in), lambda: (0, 0)),
            pl.BlockSpec((F_in, H), lambda: (0, 0)),
            pl.BlockSpec((1, H), lambda: (0, 0)),
        ],
        out_specs=pl.BlockSpec((N_PAD, H), lambda: (0, 0)),
    )(jnp.pad(x, ((0, N_PAD - N), (0, 0))), W_enc, b_enc.reshape(1, H))

    sc0 = _sc_aggr_kernel(N_PAD, E, H, gather_input=True)
    xl0, den0, num0, _ = sc0(h, row2d, col2d, t0v)

    BE = 8000
    nblk = E // BE
    wspec = lambda shape: pl.BlockSpec(shape, lambda i: (0, 0))
    espec = pl.BlockSpec((BE, H2), lambda i: (i, 0))
    xl1, hh = pl.pallas_call(
        _mid_body,
        grid=(nblk,),
        out_shape=[
            jax.ShapeDtypeStruct((E, H2), F32),
            jax.ShapeDtypeStruct((E, H2), F32),
        ],
        in_specs=[
            espec, espec, espec,
            wspec((H2, Hm)), wspec((1, Hm)), wspec((1, Hm)), wspec((1, Hm)),
            wspec((Hm, H2)), wspec((1, H2)),
            wspec((1, H2)), wspec((1, H2)),
        ],
        out_specs=[espec, espec],
    )(xl0, den0, num0, W1_0, b1_0.reshape(1, Hm), g1_0.reshape(1, Hm),
      be1_0.reshape(1, Hm), W2_0, b2_0.reshape(1, H2), n1_g.reshape(1, H2),
      n1_b.reshape(1, H2))

    sc1 = _sc_aggr_kernel(N_PAD, E, H, gather_input=False)
    _, den1, num1, _ = sc1(hh, row2d, col2d, t1v)

    y = pl.pallas_call(
        _final_body,
        grid=(nblk,),
        out_shape=jax.ShapeDtypeStruct((E, F_out), F32),
        in_specs=[
            espec, espec, espec, espec,
            wspec((H2, Hm)), wspec((1, Hm)), wspec((1, Hm)), wspec((1, Hm)),
            wspec((Hm, H2)), wspec((1, H2)),
            wspec((1, H2)), wspec((1, H2)),
            wspec((H2, F_out)), wspec((1, F_out)),
        ],
        out_specs=pl.BlockSpec((BE, F_out), lambda i: (i, 0)),
    )(hh, xl1, den1, num1, W1_1, b1_1.reshape(1, Hm), g1_1.reshape(1, Hm),
      be1_1.reshape(1, Hm), W2_1, b2_1.reshape(1, H2), n0_g.reshape(1, H2),
      n0_b.reshape(1, H2), W_lin, b_lin.reshape(1, F_out))

    return y


# speedup vs baseline: 1.2737x; 1.2737x over previous
import functools

import jax
import jax.numpy as jnp
from jax import lax
from jax.experimental import pallas as pl
from jax.experimental.pallas import tpu as pltpu
from jax.experimental.pallas import tpu_sc as plsc

F32 = jnp.float32
EPS = 1e-7
LN_EPS = 1e-5
DEN_EPS = 1e-16

NSUB = 16
LANES = 16
CH = 80


def _sc_aggr_kernel(N_PAD, E, H, gather_input):
    H2 = 2 * H
    H4 = 4 * H
    EW = E // NSUB
    NCH = EW // CH
    NCHP = ((NCH + 7) // 8) * 8
    ROWS_W = N_PAD // NSUB

    mesh = plsc.VectorSubcoreMesh(core_axis_name="c", subcore_axis_name="s",
                                  num_cores=1)

    out_type = [
        jax.ShapeDtypeStruct((E, H2), F32),
        jax.ShapeDtypeStruct((E, H2), F32),
        jax.ShapeDtypeStruct((N_PAD, H2), F32),
    ]
    scratch = [
        pltpu.VMEM((NCHP, CH), jnp.int32),
        pltpu.VMEM((NCHP, CH), jnp.int32),
        pltpu.VMEM((2, CH, H), F32),
        pltpu.VMEM((2, CH, H), F32),
        pltpu.VMEM((2, CH, H4), F32),
        pltpu.VMEM((2, CH, H2), F32),
        pltpu.VMEM((2, CH, H2), F32),
        pltpu.VMEM((CH, H4), F32),
        pltpu.VMEM((LANES,), F32),
        pltpu.VMEM_SHARED((N_PAD, H4), F32),
        pltpu.SemaphoreType.DMA((2,)),
        pltpu.SemaphoreType.DMA((2,)),
        pltpu.SemaphoreType.DMA((2,)),
        pltpu.SemaphoreType.DMA((2,)),
    ]

    @functools.partial(pl.kernel, out_type=out_type, mesh=mesh,
                       scratch_types=scratch,
                       compiler_params=pltpu.CompilerParams(
                           use_tc_tiling_on_sc=False))
    def body(h_hbm, row2d, col2d, tvec, xl0_hbm, aggr_hbm, an_hbm,
             rowv, colv, hr_v, hc_v, contrib, xlbuf,
             gbuf, zbuf, tv_v, acc, semg, semc, sems, semo):
        s = lax.axis_index("s")
        ebase = s * EW

        z16 = jnp.zeros((LANES,), F32)

        def zrow(i, carry):
            for k in range(4):
                zbuf[i, pl.ds(k * LANES, LANES)] = z16
            return carry

        lax.fori_loop(0, CH, zrow, 0, unroll=8)
        for z in range(ROWS_W // CH):
            pltpu.sync_copy(zbuf, acc.at[pl.ds(s * ROWS_W + z * CH, CH)])

        pltpu.sync_copy(row2d.at[pl.ds(s * NCHP, NCHP)], rowv)
        pltpu.sync_copy(col2d.at[pl.ds(s * NCHP, NCHP)], colv)
        pltpu.sync_copy(tvec, tv_v)
        tv = tv_v[...]

        plsc.subcore_barrier()

        def issue_in(j, p):
            if gather_input:
                pltpu.async_copy(h_hbm.at[rowv.at[j]], hr_v.at[p], semg.at[p])
                pltpu.async_copy(h_hbm.at[colv.at[j]], hc_v.at[p], semc.at[p])
            else:
                pltpu.async_copy(h_hbm.at[pl.ds(ebase + j * CH, CH)],
                                 xlbuf.at[p], semg.at[p])

        def wait_in(j, p):
            if gather_input:
                pltpu.make_async_copy(h_hbm.at[rowv.at[j]], hr_v.at[p],
                                      semg.at[p]).wait()
                pltpu.make_async_copy(h_hbm.at[colv.at[j]], hc_v.at[p],
                                      semc.at[p]).wait()
            else:
                pltpu.make_async_copy(h_hbm.at[pl.ds(ebase + j * CH, CH)],
                                      xlbuf.at[p], semg.at[p]).wait()

        def wait_scat(j, p):
            pltpu.make_async_copy(contrib.at[p], acc.at[colv.at[j]],
                                  sems.at[p]).wait()
            if gather_input:
                pltpu.make_async_copy(
                    xlbuf.at[p], xl0_hbm.at[pl.ds(ebase + j * CH, CH)],
                    semo.at[p]).wait()

        issue_in(0, 0)

        def chunk(j, carry):
            p = lax.rem(j, 2)
            q = 1 - p

            @pl.when(j + 1 < NCH)
            def _():
                issue_in(j + 1, q)

            wait_in(j, p)

            @pl.when(j >= 2)
            def _():
                wait_scat(j, p)

            def edge(e, ecarry):
                if gather_input:
                    hr = hr_v[p, e]
                    hc = hc_v[p, e]
                    mr = jnp.maximum(hr, 0.0)
                    mc = jnp.maximum(hc, 0.0)
                    xlbuf[p, e, pl.ds(0, LANES)] = hr
                    xlbuf[p, e, pl.ds(LANES, LANES)] = hc
                else:
                    mr = xlbuf[p, e, pl.ds(0, LANES)]
                    mc = xlbuf[p, e, pl.ds(LANES, LANES)]
                er = jnp.exp(mr * tv)
                ec = jnp.exp(mc * tv)
                contrib[p, e, pl.ds(0, LANES)] = er
                contrib[p, e, pl.ds(LANES, LANES)] = ec
                contrib[p, e, pl.ds(2 * LANES, LANES)] = mr * er
                contrib[p, e, pl.ds(3 * LANES, LANES)] = mc * ec
                return ecarry

            lax.fori_loop(0, CH, edge, 0, unroll=4)
            pltpu.async_copy(contrib.at[p], acc.at[colv.at[j]], sems.at[p],
                             add=True)
            if gather_input:
                pltpu.async_copy(
                    xlbuf.at[p], xl0_hbm.at[pl.ds(ebase + j * CH, CH)],
                    semo.at[p])
            return carry

        lax.fori_loop(0, NCH, chunk, 0)
        wait_scat(NCH - 2, (NCH - 2) % 2)
        wait_scat(NCH - 1, (NCH - 1) % 2)
        plsc.subcore_barrier()

        denv = jnp.full((LANES,), DEN_EPS, F32)
        for z in range(ROWS_W // CH):
            nsl = pl.ds(s * ROWS_W + z * CH, CH)
            pltpu.sync_copy(acc.at[nsl], zbuf)

            def nrow(i, carry):
                for k in range(2):
                    den = zbuf[i, pl.ds(k * LANES, LANES)]
                    num = zbuf[i, pl.ds(H2 + k * LANES, LANES)]
                    gbuf[0, i, pl.ds(k * LANES, LANES)] = num / (den + denv)
                return carry

            lax.fori_loop(0, CH, nrow, 0, unroll=8)
            pltpu.sync_copy(gbuf.at[0], an_hbm.at[nsl])
        plsc.subcore_barrier()

        def issue_g(j, p):
            pltpu.async_copy(an_hbm.at[rowv.at[j]], gbuf.at[p], semg.at[p])

        def wait_g(j, p):
            pltpu.make_async_copy(an_hbm.at[rowv.at[j]], gbuf.at[p],
                                  semg.at[p]).wait()

        def wait_out(j, p):
            pltpu.make_async_copy(
                gbuf.at[p], aggr_hbm.at[pl.ds(ebase + j * CH, CH)],
                semo.at[p]).wait()

        issue_g(0, 0)

        def g(j, carry):
            p = lax.rem(j, 2)
            q = 1 - p

            @pl.when(j >= 1)
            def _():
                wait_out(j - 1, q)

            @pl.when(j + 1 < NCH)
            def _():
                issue_g(j + 1, q)

            wait_g(j, p)
            pltpu.async_copy(gbuf.at[p],
                             aggr_hbm.at[pl.ds(ebase + j * CH, CH)],
                             semo.at[p])
            return carry

        lax.fori_loop(0, NCH, g, 0)
        wait_out(NCH - 1, (NCH - 1) % 2)

    return body


def _layer_norm_block(z, g, b):
    k = z.shape[-1]
    w = jnp.full((k, k), 1.0 / k, F32)
    mu = jnp.dot(z, w, preferred_element_type=F32)
    d = z - mu
    var = jnp.dot(d * d, w, preferred_element_type=F32)
    return d * lax.rsqrt(var + LN_EPS) * g + b


def _enc_body(x_ref, w_ref, b_ref, o_ref):
    o_ref[...] = (
        jnp.dot(x_ref[...], w_ref[...], preferred_element_type=F32)
        + b_ref[...]
    )


def _mid_body(xl0_ref, aggr_ref, w1_ref, b1_ref, g1_ref, be1_ref,
              w2_ref, b2_ref, ng_ref, nb_ref, xl1_ref, hh_ref):
    out = aggr_ref[...] + xl0_ref[...]
    z = jnp.dot(out, w1_ref[...], preferred_element_type=F32) + b1_ref[...]
    z = _layer_norm_block(z, g1_ref[...], be1_ref[...])
    z = jnp.maximum(z, 0.0)
    xl1 = jnp.dot(z, w2_ref[...], preferred_element_type=F32) + b2_ref[...]
    xl1_ref[...] = xl1
    hh_ref[...] = jnp.maximum(_layer_norm_block(xl1, ng_ref[...], nb_ref[...]),
                              0.0)


def _final_body(hh_ref, xl1_ref, aggr_ref, w1_ref, b1_ref, g1_ref,
                be1_ref, w2_ref, b2_ref, n0g_ref, n0b_ref, wl_ref, bl_ref,
                y_ref):
    out = aggr_ref[...] + hh_ref[...]
    z = jnp.dot(out, w1_ref[...], preferred_element_type=F32) + b1_ref[...]
    z = _layer_norm_block(z, g1_ref[...], be1_ref[...])
    z = jnp.maximum(z, 0.0)
    z = jnp.dot(z, w2_ref[...], preferred_element_type=F32) + b2_ref[...]
    xl2 = xl1_ref[...] + z
    q = jnp.maximum(_layer_norm_block(xl2, n0g_ref[...], n0b_ref[...]), 0.0)
    y_ref[...] = (
        jnp.dot(q, wl_ref[...], preferred_element_type=F32) + bl_ref[...]
    )


def kernel(x, edge_index, edge_attr, W_enc, b_enc, t0, W1_0, b1_0, g1_0,
           be1_0, W2_0, b2_0, t1, W1_1, b1_1, g1_1, be1_1, W2_1, b2_1,
           n0_g, n0_b, n1_g, n1_b, W_lin, b_lin):
    N, F_in = x.shape
    E = edge_index.shape[1]
    H = W_enc.shape[1]
    H2 = 2 * H
    H4 = 4 * H
    Hm = W1_0.shape[1]
    F_out = W_lin.shape[1]

    N_PAD = ((N + NSUB * CH - 1) // (NSUB * CH)) * (NSUB * CH)

    nch = (E // NSUB) // CH
    nchp = ((nch + 7) // 8) * 8

    def pad_idx(v):
        v3 = v.astype(jnp.int32).reshape(NSUB, nch, CH)
        v3 = jnp.pad(v3, ((0, 0), (0, nchp - nch), (0, 0)))
        return v3.reshape(NSUB * nchp, CH)

    row2d = pad_idx(edge_index[0])
    col2d = pad_idx(edge_index[1])
    t0v = jnp.full((LANES,), t0, F32)
    t1v = jnp.full((LANES,), t1, F32)

    h = pl.pallas_call(
        _enc_body,
        out_shape=jax.ShapeDtypeStruct((N_PAD, H), F32),
        in_specs=[
            pl.BlockSpec((N_PAD, F_in), lambda: (0, 0)),
            pl.BlockSpec((F_in, H), lambda: (0, 0)),
            pl.BlockSpec((1, H), lambda: (0, 0)),
        ],
        out_specs=pl.BlockSpec((N_PAD, H), lambda: (0, 0)),
    )(jnp.pad(x, ((0, N_PAD - N), (0, 0))), W_enc, b_enc.reshape(1, H))

    sc0 = _sc_aggr_kernel(N_PAD, E, H, gather_input=True)
    xl0, aggr0, _ = sc0(h, row2d, col2d, t0v)

    BE = 8000
    nblk = E // BE
    wspec = lambda shape: pl.BlockSpec(shape, lambda i: (0, 0))
    espec = pl.BlockSpec((BE, H2), lambda i: (i, 0))
    xl1, hh = pl.pallas_call(
        _mid_body,
        grid=(nblk,),
        out_shape=[
            jax.ShapeDtypeStruct((E, H2), F32),
            jax.ShapeDtypeStruct((E, H2), F32),
        ],
        in_specs=[
            espec, espec,
            wspec((H2, Hm)), wspec((1, Hm)), wspec((1, Hm)), wspec((1, Hm)),
            wspec((Hm, H2)), wspec((1, H2)),
            wspec((1, H2)), wspec((1, H2)),
        ],
        out_specs=[espec, espec],
    )(xl0, aggr0, W1_0, b1_0.reshape(1, Hm), g1_0.reshape(1, Hm),
      be1_0.reshape(1, Hm), W2_0, b2_0.reshape(1, H2), n1_g.reshape(1, H2),
      n1_b.reshape(1, H2))

    sc1 = _sc_aggr_kernel(N_PAD, E, H, gather_input=False)
    _, aggr1, _ = sc1(hh, row2d, col2d, t1v)

    y = pl.pallas_call(
        _final_body,
        grid=(nblk,),
        out_shape=jax.ShapeDtypeStruct((E, F_out), F32),
        in_specs=[
            espec, espec, espec,
            wspec((H2, Hm)), wspec((1, Hm)), wspec((1, Hm)), wspec((1, Hm)),
            wspec((Hm, H2)), wspec((1, H2)),
            wspec((1, H2)), wspec((1, H2)),
            wspec((H2, F_out)), wspec((1, F_out)),
        ],
        out_specs=pl.BlockSpec((BE, F_out), lambda i: (i, 0)),
    )(hh, xl1, aggr1, W1_1, b1_1.reshape(1, Hm), g1_1.reshape(1, Hm),
      be1_1.reshape(1, Hm), W2_1, b2_1.reshape(1, H2), n0_g.reshape(1, H2),
      n0_b.reshape(1, H2), W_lin, b_lin.reshape(1, F_out))

    return y
